# baseline (speedup 1.0000x reference)
import jax
import jax.numpy as jnp
from jax.experimental import pallas as pl

_N = 4096
_H = 256
_K = 15
_BLK = 512


def _sim_kernel(zb_ref, z_ref, out_ref):
    i = pl.program_id(0)
    s = jax.lax.dot_general(zb_ref[...], z_ref[...], (((1,), (1,)), ((), ())),
                            preferred_element_type=jnp.float32)

    neg = jnp.full((_BLK, 128), -jnp.inf, dtype=jnp.float32)
    m1, m2, m3, m4 = neg, neg, neg, neg
    for c in range(_N // 128):
        v = s[:, c * 128:(c + 1) * 128]
        r = jnp.minimum(m1, v)
        m1 = jnp.maximum(m1, v)
        r, m2 = jnp.minimum(m2, r), jnp.maximum(m2, r)
        r, m3 = jnp.minimum(m3, r), jnp.maximum(m3, r)
        m4 = jnp.maximum(m4, r)
    h1, h2, h3, h4 = m1, m2, m3, m4
    m = jnp.max(h1, axis=-1, keepdims=True)
    for _ in range(_K - 1):
        sel = h1 == m
        h1 = jnp.where(sel, h2, h1)
        h2 = jnp.where(sel, h3, h2)
        h3 = jnp.where(sel, h4, h3)
        h4 = jnp.where(sel, -jnp.inf, h4)
        m = jnp.max(h1, axis=-1, keepdims=True)

    eye = (jax.lax.broadcasted_iota(jnp.int32, (_BLK, _BLK), 0) ==
           jax.lax.broadcasted_iota(jnp.int32, (_BLK, _BLK), 1))

    def _store_masked(thr):
        out_ref[...] = jnp.where(s >= thr, s, 0.0)
        db = out_ref[:, pl.ds(i * _BLK, _BLK)]
        out_ref[:, pl.ds(i * _BLK, _BLK)] = jnp.where(eye, 0.0, db)

    _store_masked(m)

    def _count_mismatch():
        cnt = jnp.sum((s >= m).astype(jnp.int32), axis=-1, keepdims=True)
        return jnp.any(cnt != _K).astype(jnp.int32)

    bad = jax.lax.cond(jnp.any(m4 >= m), _count_mismatch,
                       lambda: jnp.zeros((), jnp.int32))

    @pl.when(bad != 0)
    def _exact_rewrite():
        mm = jnp.max(s, axis=-1, keepdims=True)
        for _ in range(_K - 1):
            mm = jnp.max(jnp.where(s < mm, s, -jnp.inf), axis=-1,
                         keepdims=True)
        _store_masked(mm)


def kernel(feat, W, b):
    z = feat @ W + b
    z = z / jnp.maximum(jnp.linalg.norm(z, axis=-1, keepdims=True), 1e-12)
    return pl.pallas_call(
        _sim_kernel,
        grid=(_N // _BLK,),
        in_specs=[pl.BlockSpec((_BLK, _H), lambda i: (i, 0)),
                  pl.BlockSpec((_N, _H), lambda i: (0, 0))],
        out_specs=pl.BlockSpec((_BLK, _N), lambda i: (i, 0)),
        out_shape=jax.ShapeDtypeStruct((_N, _N), jnp.float32),
    )(z, z)
